# baseline (speedup 1.0000x reference)
import functools

import jax
import jax.numpy as jnp
from jax import lax
from jax.experimental import pallas as pl
from jax.experimental.pallas import tpu as pltpu
from jax.experimental.pallas import tpu_sc as plsc

VOCAB = 100000
EMB = 128
HID = 128
CTX = 200

NC = 2
NS = 16
NW = NC * NS
BPW = 8
CTX_PAD = NW * BPW
LANES = 16

TILE = 2048
GRID = (VOCAB + TILE - 1) // TILE
VPAD = GRID * TILE


def _sc_gather_sum(idx_pad, table):
    mesh = plsc.VectorSubcoreMesh(core_axis_name="c", subcore_axis_name="s")

    @functools.partial(
        pl.kernel,
        out_type=jax.ShapeDtypeStruct((NW, EMB), jnp.float32),
        mesh=mesh,
        scratch_types=[
            pltpu.VMEM((BPW,), jnp.int32),
            pltpu.VMEM((BPW, EMB), jnp.float32),
            pltpu.VMEM((EMB,), jnp.float32),
            pltpu.SemaphoreType.DMA,
        ],
    )
    def gather_kernel(idx_hbm, table_hbm, out_hbm, idx_v, rows_v, acc_v, sem):
        wid = lax.axis_index("s") * NC + lax.axis_index("c")
        base = wid * BPW
        pltpu.sync_copy(idx_hbm.at[pl.ds(base, BPW)], idx_v)
        pltpu.async_copy(table_hbm.at[idx_v], rows_v, sem).wait()
        for c in range(EMB // LANES):
            acc = jnp.zeros((LANES,), jnp.float32)
            for j in range(BPW):
                row = rows_v[j, pl.ds(c * LANES, LANES)]
                acc = acc + jnp.where(base + j < CTX, row,
                                      jnp.zeros((LANES,), jnp.float32))
            acc_v[pl.ds(c * LANES, LANES)] = acc
        pltpu.sync_copy(acc_v, out_hbm.at[wid])

    return gather_kernel(idx_pad, table)


def _tc_body(part_ref, wp_ref, bp_ref, wo_ref, bo_ref,
             out_ref, ms_out_ref, h_ref, m_ref, s_ref):
    g = pl.program_id(0)

    @pl.when(g == 0)
    def _init():
        e = jnp.sum(part_ref[...], axis=0, keepdims=True)
        h = jnp.dot(e, wp_ref[...], preferred_element_type=jnp.float32)
        h_ref[...] = jnp.maximum(h + bp_ref[...], 0.0)
        m_ref[0] = -jnp.inf
        s_ref[0] = 0.0

    logits = jnp.dot(h_ref[...], wo_ref[...],
                     preferred_element_type=jnp.float32) + bo_ref[...]
    cols = g * TILE + lax.broadcasted_iota(jnp.int32, (1, TILE), 1)
    lm = jnp.where(cols < VOCAB, logits, -jnp.inf)
    m_old = m_ref[0]
    m_new = jnp.maximum(m_old, jnp.max(lm))
    s_ref[0] = s_ref[0] * jnp.exp(m_old - m_new) + jnp.sum(jnp.exp(lm - m_new))
    m_ref[0] = m_new
    out_ref[...] = logits

    @pl.when(g == GRID - 1)
    def _finish():
        ms_out_ref[0] = m_ref[0] + jnp.log(s_ref[0])


def _tc_mlp_logits(partials, W_proj, b_proj2d, W_out, b_out2d):
    return pl.pallas_call(
        _tc_body,
        grid=(GRID,),
        in_specs=[
            pl.BlockSpec((NW, EMB), lambda g: (0, 0)),
            pl.BlockSpec((EMB, HID), lambda g: (0, 0)),
            pl.BlockSpec((1, HID), lambda g: (0, 0)),
            pl.BlockSpec((HID, TILE), lambda g: (0, g)),
            pl.BlockSpec((1, TILE), lambda g: (0, g)),
        ],
        out_specs=[
            pl.BlockSpec((1, TILE), lambda g: (0, g)),
            pl.BlockSpec(memory_space=pltpu.SMEM),
        ],
        out_shape=[
            jax.ShapeDtypeStruct((1, VPAD), jnp.float32),
            jax.ShapeDtypeStruct((1,), jnp.float32),
        ],
        scratch_shapes=[
            pltpu.VMEM((1, HID), jnp.float32),
            pltpu.SMEM((1,), jnp.float32),
            pltpu.SMEM((1,), jnp.float32),
        ],
    )(partials, W_proj, b_proj2d, W_out, b_out2d)


def _norm_body(lg_ref, lse_ref, out_ref):
    out_ref[...] = lg_ref[...] - lse_ref[0]


def _tc_normalize(logits2d, lse):
    return pl.pallas_call(
        _norm_body,
        in_specs=[
            pl.BlockSpec((GRID, TILE), lambda: (0, 0)),
            pl.BlockSpec(memory_space=pltpu.SMEM),
        ],
        out_specs=pl.BlockSpec((GRID, TILE), lambda: (0, 0)),
        out_shape=jax.ShapeDtypeStruct((GRID, TILE), jnp.float32),
    )(logits2d, lse)


def kernel(inputs, table, W_proj, b_proj, W_out, b_out):
    idx_pad = jnp.zeros((CTX_PAD,), jnp.int32).at[:CTX].set(
        inputs.astype(jnp.int32))
    partials = _sc_gather_sum(idx_pad, table)
    logits, lse = _tc_mlp_logits(partials, W_proj, b_proj.reshape(1, HID),
                                 W_out, b_out.reshape(1, VOCAB))
    out = _tc_normalize(logits.reshape(GRID, TILE), lse)
    return out.reshape(1, VPAD)[:, :VOCAB]


# speedup vs baseline: 1.2238x; 1.2238x over previous
import functools

import jax
import jax.numpy as jnp
from jax import lax
from jax.experimental import pallas as pl
from jax.experimental.pallas import tpu as pltpu
from jax.experimental.pallas import tpu_sc as plsc

VOCAB = 100000
EMB = 128
HID = 128
CTX = 200

NC = 2
NS = 16
NW = NC * NS
BPW = 8
CTX_PAD = NW * BPW
LANES = 16

TILE = 4096
GRID = (VOCAB + TILE - 1) // TILE
VPAD = GRID * TILE


def _sc_gather_sum(idx_pad, table):
    mesh = plsc.VectorSubcoreMesh(core_axis_name="c", subcore_axis_name="s")

    @functools.partial(
        pl.kernel,
        out_type=jax.ShapeDtypeStruct((NW, EMB), jnp.float32),
        mesh=mesh,
        scratch_types=[
            pltpu.VMEM((BPW,), jnp.int32),
            pltpu.VMEM((BPW, EMB), jnp.float32),
            pltpu.VMEM((EMB,), jnp.float32),
            pltpu.SemaphoreType.DMA,
        ],
    )
    def gather_kernel(idx_hbm, table_hbm, out_hbm, idx_v, rows_v, acc_v, sem):
        wid = lax.axis_index("s") * NC + lax.axis_index("c")
        base = wid * BPW
        pltpu.sync_copy(idx_hbm.at[pl.ds(base, BPW)], idx_v)
        pltpu.async_copy(table_hbm.at[idx_v], rows_v, sem).wait()
        for c in range(EMB // LANES):
            acc = jnp.zeros((LANES,), jnp.float32)
            for j in range(BPW):
                row = rows_v[j, pl.ds(c * LANES, LANES)]
                acc = acc + jnp.where(base + j < CTX, row,
                                      jnp.zeros((LANES,), jnp.float32))
            acc_v[pl.ds(c * LANES, LANES)] = acc
        pltpu.sync_copy(acc_v, out_hbm.at[wid])

    return gather_kernel(idx_pad, table)


def _tc_body(part_ref, wp_ref, bp_ref, wo_ref, bo_ref,
             out_ref, h_ref, m_ref, s_ref):
    g = pl.program_id(0)

    @pl.when(g == 0)
    def _init():
        e = jnp.sum(part_ref[...], axis=0, keepdims=True)
        h = jnp.dot(e, wp_ref[...], preferred_element_type=jnp.float32)
        h_ref[...] = jnp.maximum(h + bp_ref[...], 0.0)
        m_ref[0] = -jnp.inf
        s_ref[0] = 0.0

    logits = jnp.dot(h_ref[...], wo_ref[...],
                     preferred_element_type=jnp.float32) + bo_ref[...]
    cols = g * TILE + lax.broadcasted_iota(jnp.int32, (1, TILE), 1)
    lm = jnp.where(cols < VOCAB, logits, -jnp.inf)
    m_old = m_ref[0]
    m_new = jnp.maximum(m_old, jnp.max(lm))
    s_ref[0] = s_ref[0] * jnp.exp(m_old - m_new) + jnp.sum(jnp.exp(lm - m_new))
    m_ref[0] = m_new
    out_ref[0, pl.ds(g * TILE, TILE)] = logits[0]

    @pl.when(g == GRID - 1)
    def _finish():
        out_ref[...] = out_ref[...] - (m_ref[0] + jnp.log(s_ref[0]))


def _tc_mlp_logits(partials, W_proj, b_proj2d, W_out, b_out2d):
    return pl.pallas_call(
        _tc_body,
        grid=(GRID,),
        in_specs=[
            pl.BlockSpec((NW, EMB), lambda g: (0, 0)),
            pl.BlockSpec((EMB, HID), lambda g: (0, 0)),
            pl.BlockSpec((1, HID), lambda g: (0, 0)),
            pl.BlockSpec((HID, TILE), lambda g: (0, g)),
            pl.BlockSpec((1, TILE), lambda g: (0, g)),
        ],
        out_specs=pl.BlockSpec((1, VPAD), lambda g: (0, 0)),
        out_shape=jax.ShapeDtypeStruct((1, VPAD), jnp.float32),
        scratch_shapes=[
            pltpu.VMEM((1, HID), jnp.float32),
            pltpu.SMEM((1,), jnp.float32),
            pltpu.SMEM((1,), jnp.float32),
        ],
    )(partials, W_proj, b_proj2d, W_out, b_out2d)


def kernel(inputs, table, W_proj, b_proj, W_out, b_out):
    idx_pad = jnp.zeros((CTX_PAD,), jnp.int32).at[:CTX].set(
        inputs.astype(jnp.int32))
    partials = _sc_gather_sum(idx_pad, table)
    out = _tc_mlp_logits(partials, W_proj, b_proj.reshape(1, HID),
                         W_out, b_out.reshape(1, VOCAB))
    return out[:, :VOCAB]


# speedup vs baseline: 1.3369x; 1.0924x over previous
import functools

import jax
import jax.numpy as jnp
from jax import lax
from jax.experimental import pallas as pl
from jax.experimental.pallas import tpu as pltpu
from jax.experimental.pallas import tpu_sc as plsc

VOCAB = 100000
EMB = 128
HID = 128
CTX = 200

NC = 2
NS = 16
NW = NC * NS
BPW = 8
CTX_PAD = NW * BPW
LANES = 16

TILE = 8192
GRID = (VOCAB + TILE - 1) // TILE
VPAD = GRID * TILE


def _sc_gather_sum(idx_pad, table):
    mesh = plsc.VectorSubcoreMesh(core_axis_name="c", subcore_axis_name="s")

    @functools.partial(
        pl.kernel,
        out_type=jax.ShapeDtypeStruct((NW, EMB), jnp.float32),
        mesh=mesh,
        scratch_types=[
            pltpu.VMEM((BPW,), jnp.int32),
            pltpu.VMEM((BPW, EMB), jnp.float32),
            pltpu.VMEM((EMB,), jnp.float32),
            pltpu.SemaphoreType.DMA,
        ],
    )
    def gather_kernel(idx_hbm, table_hbm, out_hbm, idx_v, rows_v, acc_v, sem):
        wid = lax.axis_index("s") * NC + lax.axis_index("c")
        base = wid * BPW
        pltpu.sync_copy(idx_hbm.at[pl.ds(base, BPW)], idx_v)
        pltpu.async_copy(table_hbm.at[idx_v], rows_v, sem).wait()
        for c in range(EMB // LANES):
            acc = jnp.zeros((LANES,), jnp.float32)
            for j in range(BPW):
                row = rows_v[j, pl.ds(c * LANES, LANES)]
                acc = acc + jnp.where(base + j < CTX, row,
                                      jnp.zeros((LANES,), jnp.float32))
            acc_v[pl.ds(c * LANES, LANES)] = acc
        pltpu.sync_copy(acc_v, out_hbm.at[wid])

    return gather_kernel(idx_pad, table)


def _tc_body(part_ref, wp_ref, bp_ref, wo_ref, bo_ref,
             out_ref, h_ref, m_ref, s_ref):
    g = pl.program_id(0)

    @pl.when(g == 0)
    def _init():
        e = jnp.sum(part_ref[...], axis=0, keepdims=True)
        h = jnp.dot(e, wp_ref[...], preferred_element_type=jnp.float32)
        h_ref[...] = jnp.maximum(h + bp_ref[...], 0.0)
        m_ref[0] = -jnp.inf
        s_ref[0] = 0.0

    logits = jnp.dot(h_ref[...], wo_ref[...],
                     preferred_element_type=jnp.float32) + bo_ref[...]
    cols = g * TILE + lax.broadcasted_iota(jnp.int32, (1, TILE), 1)
    lm = jnp.where(cols < VOCAB, logits, -jnp.inf)
    m_old = m_ref[0]
    m_new = jnp.maximum(m_old, jnp.max(lm))
    s_ref[0] = s_ref[0] * jnp.exp(m_old - m_new) + jnp.sum(jnp.exp(lm - m_new))
    m_ref[0] = m_new
    out_ref[0, pl.ds(g * TILE, TILE)] = logits[0]

    @pl.when(g == GRID - 1)
    def _finish():
        out_ref[...] = out_ref[...] - (m_ref[0] + jnp.log(s_ref[0]))


def _tc_mlp_logits(partials, W_proj, b_proj2d, W_out, b_out2d):
    return pl.pallas_call(
        _tc_body,
        grid=(GRID,),
        in_specs=[
            pl.BlockSpec((NW, EMB), lambda g: (0, 0)),
            pl.BlockSpec((EMB, HID), lambda g: (0, 0)),
            pl.BlockSpec((1, HID), lambda g: (0, 0)),
            pl.BlockSpec((HID, TILE), lambda g: (0, g)),
            pl.BlockSpec((1, TILE), lambda g: (0, g)),
        ],
        out_specs=pl.BlockSpec((1, VPAD), lambda g: (0, 0)),
        out_shape=jax.ShapeDtypeStruct((1, VPAD), jnp.float32),
        scratch_shapes=[
            pltpu.VMEM((1, HID), jnp.float32),
            pltpu.SMEM((1,), jnp.float32),
            pltpu.SMEM((1,), jnp.float32),
        ],
    )(partials, W_proj, b_proj2d, W_out, b_out2d)


def kernel(inputs, table, W_proj, b_proj, W_out, b_out):
    idx_pad = jnp.zeros((CTX_PAD,), jnp.int32).at[:CTX].set(
        inputs.astype(jnp.int32))
    partials = _sc_gather_sum(idx_pad, table)
    out = _tc_mlp_logits(partials, W_proj, b_proj.reshape(1, HID),
                         W_out, b_out.reshape(1, VOCAB))
    return out[:, :VOCAB]
